# baseline (speedup 1.0000x reference)
import jax
import jax.numpy as jnp
from jax.experimental import pallas as pl
from jax.experimental.pallas import tpu as pltpu

BM = 400


def _lin_kernel(x_ref, w_ref, b_ref, o_ref):
    o_ref[...] = (
        jnp.dot(x_ref[...], w_ref[...], preferred_element_type=jnp.float32)
        + b_ref[...]
    )


def _pass1_kernel(adj_ref, y1_ref, w2_ref, b2_ref, o_ref):
    h = jnp.maximum(
        jnp.dot(adj_ref[...], y1_ref[...], preferred_element_type=jnp.float32),
        0.0,
    )
    o_ref[...] = (
        jnp.dot(h, w2_ref[...], preferred_element_type=jnp.float32)
        + b2_ref[...]
    )


def _pass2_kernel(adj_ref, y2_ref, o_ref):
    o_ref[...] = jnp.dot(
        adj_ref[...], y2_ref[...], preferred_element_type=jnp.float32
    )


def kernel(x, adj_t, W1, b1, W2, b2):
    n, d_in = x.shape
    d_h = W1.shape[1]
    d_out = W2.shape[1]
    b1r = b1.reshape(1, d_h)
    b2r = b2.reshape(1, d_out)

    y1 = pl.pallas_call(
        _lin_kernel,
        grid=(n // BM,),
        in_specs=[
            pl.BlockSpec((BM, d_in), lambda m: (m, 0)),
            pl.BlockSpec((d_in, d_h), lambda m: (0, 0)),
            pl.BlockSpec((1, d_h), lambda m: (0, 0)),
        ],
        out_specs=pl.BlockSpec((BM, d_h), lambda m: (m, 0)),
        out_shape=jax.ShapeDtypeStruct((n, d_h), jnp.float32),
    )(x, W1, b1r)

    y2 = pl.pallas_call(
        _pass1_kernel,
        grid=(n // BM,),
        in_specs=[
            pl.BlockSpec((BM, n), lambda m: (m, 0)),
            pl.BlockSpec((n, d_h), lambda m: (0, 0)),
            pl.BlockSpec((d_h, d_out), lambda m: (0, 0)),
            pl.BlockSpec((1, d_out), lambda m: (0, 0)),
        ],
        out_specs=pl.BlockSpec((BM, d_out), lambda m: (m, 0)),
        out_shape=jax.ShapeDtypeStruct((n, d_out), jnp.float32),
        compiler_params=pltpu.CompilerParams(
            dimension_semantics=("arbitrary",),
        ),
    )(adj_t, y1, W2, b2r)

    out = pl.pallas_call(
        _pass2_kernel,
        grid=(n // BM,),
        in_specs=[
            pl.BlockSpec((BM, n), lambda m: (m, 0)),
            pl.BlockSpec((n, d_out), lambda m: (0, 0)),
        ],
        out_specs=pl.BlockSpec((BM, d_out), lambda m: (m, 0)),
        out_shape=jax.ShapeDtypeStruct((n, d_out), jnp.float32),
        compiler_params=pltpu.CompilerParams(
            dimension_semantics=("arbitrary",),
        ),
    )(adj_t, y2)

    return out
